# baseline (speedup 1.0000x reference)
import jax
import jax.numpy as jnp
from jax import lax
from jax.experimental import pallas as pl
from jax.experimental.pallas import tpu as pltpu
from jax.experimental.pallas import tpu_sc as plsc

_N, _D, _S = 32768, 256, 128
_G = 8
_SB = 4096
_NSTEP = _N // _SB
_LANES = 16
_CH = _D // _LANES
_NC, _NS = 2, 16
_NW = _NC * _NS
_SEGW = _S // _NW


def _sc_edges_body(x_hbm, starts_hbm, ends_hbm, part_hbm,
                   starts_v, ends_v, xb_v, outb_v, sem):
    wid = lax.axis_index("s") * _NC + lax.axis_index("c")
    pltpu.sync_copy(starts_hbm, starts_v.at[pl.ds(0, _S)])
    pltpu.sync_copy(ends_hbm, ends_v.at[pl.ds(0, _S)])
    segs = []
    copies = []
    for t in range(_SEGW):
        j = wid * _SEGW + t
        s = starts_v[pl.ds(j, _LANES)][0]
        e = ends_v[pl.ds(j, _LANES)][0]
        bs = s // _G
        be = e // _G
        copies.append(pltpu.async_copy(
            x_hbm.at[pl.ds(bs * _G, _G)], xb_v.at[pl.ds(2 * t * _G, _G)],
            sem))
        copies.append(pltpu.async_copy(
            x_hbm.at[pl.ds(be * _G, _G)], xb_v.at[pl.ds((2 * t + 1) * _G, _G)],
            sem))
        segs.append((s, e, bs, be))
    for cpy in copies:
        cpy.wait()
    for t in range(_SEGW):
        s, e, bs, be = segs[t]
        rs = s - bs * _G
        re = e - be * _G
        f = (bs < be).astype(jnp.float32)
        w_s = [((u >= rs).astype(jnp.float32)
                * jnp.maximum(f, (u <= re).astype(jnp.float32)))
               for u in range(_G)]
        w_e = [f * (u <= re).astype(jnp.float32) for u in range(_G)]
        for ch in range(_CH):
            sl = pl.ds(ch * _LANES, _LANES)
            acc = jnp.zeros((_LANES,), jnp.float32)
            for u in range(_G):
                acc = acc + xb_v[2 * t * _G + u, sl] * w_s[u]
                acc = acc + xb_v[(2 * t + 1) * _G + u, sl] * w_e[u]
            outb_v[t, sl] = acc
    pltpu.sync_copy(outb_v, part_hbm.at[pl.ds(wid * _SEGW, _SEGW)])


def _sc_edges(x, starts, ends):
    mesh = plsc.VectorSubcoreMesh(core_axis_name="c", subcore_axis_name="s")
    return pl.kernel(
        _sc_edges_body,
        out_type=jax.ShapeDtypeStruct((_S, _D), jnp.float32),
        mesh=mesh,
        scratch_types=[
            pltpu.VMEM((_S + _LANES,), jnp.int32),
            pltpu.VMEM((_S + _LANES,), jnp.int32),
            pltpu.VMEM((2 * _SEGW * _G, _D), jnp.float32),
            pltpu.VMEM((_SEGW, _D), jnp.float32),
            pltpu.SemaphoreType.DMA,
        ],
    )(x, starts, ends)


def _mm_body(se_ref, x_ref, acc_out_ref, acc_ref):
    i = pl.program_id(0)

    @pl.when(i == 0)
    def _():
        acc_ref[...] = jnp.zeros_like(acc_ref)

    rows = lax.broadcasted_iota(jnp.int32, (_SB, _S), 0) + i * _SB
    s = se_ref[0:1, :]
    e = se_ref[1:2, :]
    lo = (s // _G + 1) * _G
    hi = (e // _G) * _G
    m = ((rows >= lo) & (rows < hi)).astype(jnp.float32)
    acc_ref[...] += jnp.dot(m.T, x_ref[...],
                            preferred_element_type=jnp.float32)

    @pl.when(i == _NSTEP - 1)
    def _():
        acc_out_ref[...] = acc_ref[...]


def _mm_interior(se, x):
    return pl.pallas_call(
        _mm_body,
        grid=(_NSTEP,),
        in_specs=[
            pl.BlockSpec((8, _S), lambda i: (0, 0)),
            pl.BlockSpec((_SB, _D), lambda i: (i, 0)),
        ],
        out_specs=pl.BlockSpec((_S, _D), lambda i: (0, 0)),
        out_shape=jax.ShapeDtypeStruct((_S, _D), jnp.float32),
        scratch_shapes=[pltpu.VMEM((_S, _D), jnp.float32)],
    )(se, x)


def _combine_body(se_ref, acc_ref, part_ref, out_ref):
    s = se_ref[0:1, :]
    e = se_ref[1:2, :]
    cnt = (e - s + 1).astype(jnp.float32)
    out_ref[...] = ((acc_ref[...] + part_ref[...]) / cnt.reshape(_S, 1)
                    + 0.006)


def _combine(se, acc, part):
    return pl.pallas_call(
        _combine_body,
        out_shape=jax.ShapeDtypeStruct((_S, _D), jnp.float32),
    )(se, acc, part)


@jax.jit
def kernel(input, graph):
    starts = graph[:, 0].astype(jnp.int32)
    ends = graph[:, 1].astype(jnp.int32)
    se = jnp.zeros((8, _S), jnp.int32)
    se = se.at[0].set(starts)
    se = se.at[1].set(ends)
    part = _sc_edges(input, starts, ends)
    acc = _mm_interior(se, input)
    return _combine(se, acc, part)


# speedup vs baseline: 1.0519x; 1.0519x over previous
import jax
import jax.numpy as jnp
from jax import lax
from jax.experimental import pallas as pl
from jax.experimental.pallas import tpu as pltpu
from jax.experimental.pallas import tpu_sc as plsc

_N, _D, _S = 32768, 256, 128
_G = 8
_SB = 8192
_NSTEP = _N // _SB
_LANES = 16
_CH = _D // _LANES
_NC, _NS = 2, 16
_NW = _NC * _NS
_SEGW = _S // _NW


def _sc_edges_body(x_hbm, se_hbm, part_hbm,
                   starts_v, ends_v, xb_v, outb_v, sem):
    wid = lax.axis_index("s") * _NC + lax.axis_index("c")
    pltpu.sync_copy(se_hbm.at[0], starts_v.at[pl.ds(0, _S)])
    pltpu.sync_copy(se_hbm.at[1], ends_v.at[pl.ds(0, _S)])
    segs = []
    copies = []
    for t in range(_SEGW):
        j = wid * _SEGW + t
        s = starts_v[pl.ds(j, _LANES)][0]
        e = ends_v[pl.ds(j, _LANES)][0]
        bs = s // _G
        be = e // _G
        copies.append(pltpu.async_copy(
            x_hbm.at[pl.ds(bs * _G, _G)], xb_v.at[pl.ds(2 * t * _G, _G)],
            sem))
        copies.append(pltpu.async_copy(
            x_hbm.at[pl.ds(be * _G, _G)], xb_v.at[pl.ds((2 * t + 1) * _G, _G)],
            sem))
        segs.append((s, e, bs, be))
    for cpy in copies:
        cpy.wait()
    for t in range(_SEGW):
        s, e, bs, be = segs[t]
        rs = s - bs * _G
        re = e - be * _G
        f = (bs < be).astype(jnp.float32)
        w_s = [((u >= rs).astype(jnp.float32)
                * jnp.maximum(f, (u <= re).astype(jnp.float32)))
               for u in range(_G)]
        w_e = [f * (u <= re).astype(jnp.float32) for u in range(_G)]
        for ch in range(_CH):
            sl = pl.ds(ch * _LANES, _LANES)
            acc = jnp.zeros((_LANES,), jnp.float32)
            for u in range(_G):
                acc = acc + xb_v[2 * t * _G + u, sl] * w_s[u]
                acc = acc + xb_v[(2 * t + 1) * _G + u, sl] * w_e[u]
            outb_v[t, sl] = acc
    pltpu.sync_copy(outb_v, part_hbm.at[pl.ds(wid * _SEGW, _SEGW)])


def _sc_edges(x, se):
    mesh = plsc.VectorSubcoreMesh(core_axis_name="c", subcore_axis_name="s")
    return pl.kernel(
        _sc_edges_body,
        out_type=jax.ShapeDtypeStruct((_S, _D), jnp.float32),
        mesh=mesh,
        scratch_types=[
            pltpu.VMEM((_S + _LANES,), jnp.int32),
            pltpu.VMEM((_S + _LANES,), jnp.int32),
            pltpu.VMEM((2 * _SEGW * _G, _D), jnp.float32),
            pltpu.VMEM((_SEGW, _D), jnp.float32),
            pltpu.SemaphoreType.DMA,
        ],
    )(x, se)


def _mm_body(se_ref, x_ref, acc_out_ref, acc_ref):
    i = pl.program_id(0)

    @pl.when(i == 0)
    def _():
        acc_ref[...] = jnp.zeros_like(acc_ref)

    rows = lax.broadcasted_iota(jnp.int32, (_SB, _S), 0) + i * _SB
    s = se_ref[0:1, :]
    e = se_ref[1:2, :]
    lo = (s // _G + 1) * _G
    hi = (e // _G) * _G
    m = ((rows >= lo) & (rows < hi)).astype(jnp.float32)
    acc_ref[...] += jnp.dot(m.T, x_ref[...],
                            preferred_element_type=jnp.float32)

    @pl.when(i == _NSTEP - 1)
    def _():
        acc_out_ref[...] = acc_ref[...]


def _mm_interior(se, x):
    return pl.pallas_call(
        _mm_body,
        grid=(_NSTEP,),
        in_specs=[
            pl.BlockSpec((8, _S), lambda i: (0, 0)),
            pl.BlockSpec((_SB, _D), lambda i: (i, 0)),
        ],
        out_specs=pl.BlockSpec((_S, _D), lambda i: (0, 0)),
        out_shape=jax.ShapeDtypeStruct((_S, _D), jnp.float32),
        scratch_shapes=[pltpu.VMEM((_S, _D), jnp.float32)],
    )(se, x)


def _combine_body(se_ref, acc_ref, part_ref, out_ref):
    s = se_ref[0:1, :]
    e = se_ref[1:2, :]
    cnt = (e - s + 1).astype(jnp.float32)
    out_ref[...] = ((acc_ref[...] + part_ref[...]) / cnt.reshape(_S, 1)
                    + 0.006)


def _combine(se, acc, part):
    return pl.pallas_call(
        _combine_body,
        out_shape=jax.ShapeDtypeStruct((_S, _D), jnp.float32),
    )(se, acc, part)


@jax.jit
def kernel(input, graph):
    se = jnp.pad(graph.astype(jnp.int32).T, ((0, 6), (0, 0)))
    part = _sc_edges(input, se)
    acc = _mm_interior(se, input)
    return _combine(se, acc, part)
